# Initial kernel scaffold; baseline (speedup 1.0000x reference)
import jax
import jax.numpy as jnp
from jax.experimental import pallas as pl


def kernel(x, bins, table):
    raise NotImplementedError("write your pallas kernel here")


# speedup vs baseline: 77.6515x; 77.6515x over previous
import functools

import jax
import jax.numpy as jnp
from jax import lax
from jax.experimental import pallas as pl
from jax.experimental.pallas import tpu as pltpu
from jax.experimental.pallas import tpu_sc as plsc

MIN_VALUE = -4.0
MAX_VALUE = 4.0
N_BINS = 2048
EMBED_DIMS = 64
TOTAL = 4096 * 100

NUM_WORKERS = 32
PER_TILE = TOTAL // NUM_WORKERS
CHUNK = 128
NCHUNKS = PER_TILE // CHUNK
LANES = 16
NVEC = PER_TILE // LANES

INV_STEP = (N_BINS - 2) / (MAX_VALUE - MIN_VALUE)
MAX_K = float(N_BINS - 3)


def _sc_body(x_hbm, bins_hbm, table_hbm, out_hbm, bins_v, x_v, idx_v, rows_v, gsem):
    wid = lax.axis_index("s") * 2 + lax.axis_index("c")
    base = wid * PER_TILE

    pltpu.sync_copy(bins_hbm, bins_v)
    pltpu.sync_copy(x_hbm.at[pl.ds(base, PER_TILE)], x_v)

    def compute(i, _):
        xv = x_v[pl.ds(i * LANES, LANES)]
        t = (xv - MIN_VALUE) * INV_STEP
        t = jnp.minimum(jnp.maximum(t, 0.0), MAX_K)
        k = t.astype(jnp.int32)
        bk = plsc.load_gather(bins_v, [k])
        bk1 = plsc.load_gather(bins_v, [k + 1])
        idx = k + (xv > bk).astype(jnp.int32) + (xv > bk1).astype(jnp.int32)
        idx_v[pl.ds(i * LANES, LANES)] = idx
        return 0

    lax.fori_loop(0, NVEC, compute, 0)

    def chunk(j, _):
        cp = pltpu.async_copy(
            table_hbm.at[idx_v.at[pl.ds(j * CHUNK, CHUNK)]], rows_v, gsem
        )
        cp.wait()
        pltpu.sync_copy(rows_v, out_hbm.at[pl.ds(base + j * CHUNK, CHUNK)])
        return 0

    lax.fori_loop(0, NCHUNKS, chunk, 0)


@jax.jit
def _embed_lookup(x_flat, bins_pad, table):
    mesh = plsc.VectorSubcoreMesh(core_axis_name="c", subcore_axis_name="s")
    return pl.kernel(
        _sc_body,
        out_type=jax.ShapeDtypeStruct((TOTAL, EMBED_DIMS), jnp.float32),
        mesh=mesh,
        compiler_params=pltpu.CompilerParams(
            needs_layout_passes=False, use_tc_tiling_on_sc=False
        ),
        scratch_types=[
            pltpu.VMEM((N_BINS,), jnp.float32),
            pltpu.VMEM((PER_TILE,), jnp.float32),
            pltpu.VMEM((PER_TILE,), jnp.int32),
            pltpu.VMEM((CHUNK, EMBED_DIMS), jnp.float32),
            pltpu.SemaphoreType.DMA,
        ],
    )(x_flat, bins_pad, table)


def kernel(x, bins, table):
    x_flat = x.reshape(-1)
    bins_pad = jnp.concatenate([bins, bins[-1:]])
    out = _embed_lookup(x_flat, bins_pad, table)
    return out.reshape(x.shape[0], x.shape[1], EMBED_DIMS)


# speedup vs baseline: 82.0056x; 1.0561x over previous
import functools

import jax
import jax.numpy as jnp
from jax import lax
from jax.experimental import pallas as pl
from jax.experimental.pallas import tpu as pltpu
from jax.experimental.pallas import tpu_sc as plsc

MIN_VALUE = -4.0
MAX_VALUE = 4.0
N_BINS = 2048
EMBED_DIMS = 64
TOTAL = 4096 * 100

NUM_WORKERS = 32
PER_TILE = TOTAL // NUM_WORKERS
CHUNK = 128
NCHUNKS = PER_TILE // CHUNK
LANES = 16
NVEC = PER_TILE // LANES

INV_STEP = (N_BINS - 2) / (MAX_VALUE - MIN_VALUE)
MAX_K = float(N_BINS - 3)


SPC = 5
NSUPER = NCHUNKS // SPC


def _sc_body(x_hbm, bins_hbm, table_hbm, out_hbm, bins_v, x_v, idx_v, rows_v,
             gsem, wsem):
    wid = lax.axis_index("s") * 2 + lax.axis_index("c")
    base = wid * PER_TILE

    pltpu.sync_copy(bins_hbm, bins_v)
    pltpu.sync_copy(x_hbm.at[pl.ds(base, PER_TILE)], x_v)

    def compute(i, _):
        xv = x_v[pl.ds(i * LANES, LANES)]
        t = (xv - MIN_VALUE) * INV_STEP
        t = jnp.minimum(jnp.maximum(t, 0.0), MAX_K)
        k = t.astype(jnp.int32)
        bk = plsc.load_gather(bins_v, [k])
        bk1 = plsc.load_gather(bins_v, [k + 1])
        idx = k + (xv > bk).astype(jnp.int32) + (xv > bk1).astype(jnp.int32)
        idx_v[pl.ds(i * LANES, LANES)] = idx
        return 0

    w_desc = [None, None]
    for s in range(NSUPER):
        p = s % 2
        lax.fori_loop(s * SPC * CHUNK // LANES, (s + 1) * SPC * CHUNK // LANES,
                      compute, 0)
        if w_desc[p] is not None:
            for cp in w_desc[p]:
                cp.wait()
        g_desc = [
            pltpu.async_copy(
                table_hbm.at[idx_v.at[pl.ds((s * SPC + i) * CHUNK, CHUNK)]],
                rows_v.at[p, i], gsem)
            for i in range(SPC)
        ]
        for cp in g_desc:
            cp.wait()
        w_desc[p] = [
            pltpu.async_copy(
                rows_v.at[p, i],
                out_hbm.at[pl.ds(base + (s * SPC + i) * CHUNK, CHUNK)], wsem)
            for i in range(SPC)
        ]
    for p in range(2):
        for cp in w_desc[p]:
            cp.wait()


@jax.jit
def _embed_lookup(x_flat, bins_pad, table):
    mesh = plsc.VectorSubcoreMesh(core_axis_name="c", subcore_axis_name="s")
    return pl.kernel(
        _sc_body,
        out_type=jax.ShapeDtypeStruct((TOTAL, EMBED_DIMS), jnp.float32),
        mesh=mesh,
        compiler_params=pltpu.CompilerParams(
            needs_layout_passes=False, use_tc_tiling_on_sc=False
        ),
        scratch_types=[
            pltpu.VMEM((N_BINS,), jnp.float32),
            pltpu.VMEM((PER_TILE,), jnp.float32),
            pltpu.VMEM((PER_TILE,), jnp.int32),
            pltpu.VMEM((2, SPC, CHUNK, EMBED_DIMS), jnp.float32),
            pltpu.SemaphoreType.DMA,
            pltpu.SemaphoreType.DMA,
        ],
    )(x_flat, bins_pad, table)


def kernel(x, bins, table):
    x_flat = x.reshape(-1)
    bins_pad = jnp.concatenate([bins, bins[-1:]])
    out = _embed_lookup(x_flat, bins_pad, table)
    return out.reshape(x.shape[0], x.shape[1], EMBED_DIMS)


# speedup vs baseline: 93.9917x; 1.1462x over previous
import jax
import jax.numpy as jnp
from jax import lax
from jax.experimental import pallas as pl
from jax.experimental.pallas import tpu as pltpu
from jax.experimental.pallas import tpu_sc as plsc

MIN_VALUE = -4.0
MAX_VALUE = 4.0
N_BINS = 2048
EMBED_DIMS = 64
N_ROWS = 4096
N_COLS = 100
LANES = 16

K_BLOCKS = 2
K_PER = EMBED_DIMS // K_BLOCKS
I_BLOCKS = 16
I_PER = N_ROWS // I_BLOCKS
GROUPS = I_PER // LANES

INV_STEP = (N_BINS - 2) / (MAX_VALUE - MIN_VALUE)
MAX_K = float(N_BINS - 3)


def _sc_body_final(xt_hbm, bins_hbm, tabt_hbm, out_hbm, bins_v, tab_v, x_v,
                   stage_v, sem0, sem1):
    wid = lax.axis_index("s") * 2 + lax.axis_index("c")
    ib = wid // K_BLOCKS
    kb = wid % K_BLOCKS
    i0 = ib * I_PER
    k0 = kb * K_PER

    pltpu.sync_copy(bins_hbm, bins_v)
    pltpu.sync_copy(tabt_hbm.at[pl.ds(k0, K_PER), :], tab_v)
    pltpu.sync_copy(xt_hbm.at[:, pl.ds(i0, I_PER)], x_v)

    sems = (sem0, sem1)

    def column(j, p):
        def group(g, _):
            xv = x_v[j, pl.ds(g * LANES, LANES)]
            t = (xv - MIN_VALUE) * INV_STEP
            t = jnp.minimum(jnp.maximum(t, 0.0), MAX_K)
            kk = t.astype(jnp.int32)
            bk = plsc.load_gather(bins_v, [kk])
            bk1 = plsc.load_gather(bins_v, [kk + 1])
            idx = kk + (xv > bk).astype(jnp.int32) + (xv > bk1).astype(jnp.int32)
            for k in range(K_PER):
                val = plsc.load_gather(tab_v,
                                       [jnp.full((LANES,), k, jnp.int32), idx])
                stage_v[p, k, pl.ds(g * LANES, LANES)] = val
            return 0

        lax.fori_loop(0, GROUPS, group, 0)

    def write(j, p):
        return pltpu.async_copy(
            stage_v.at[p], out_hbm.at[j, pl.ds(k0, K_PER), pl.ds(i0, I_PER)],
            sems[p])

    column(0, 0)
    cp0 = write(0, 0)
    column(1, 1)
    cp1 = write(1, 1)

    def step(j2, _):
        j = j2 * 2 + 2
        cp0.wait()
        column(j, 0)
        write(j, 0)
        cp1.wait()
        column(j + 1, 1)
        write(j + 1, 1)
        return 0

    lax.fori_loop(0, (N_COLS - 2) // 2, step, 0)
    cp0.wait()
    cp1.wait()


@jax.jit
def _embed_lookup(xt, bins_pad, tabt):
    mesh = plsc.VectorSubcoreMesh(core_axis_name="c", subcore_axis_name="s")
    return pl.kernel(
        _sc_body_final,
        out_type=jax.ShapeDtypeStruct((N_COLS, EMBED_DIMS, N_ROWS), jnp.float32),
        mesh=mesh,
        compiler_params=pltpu.CompilerParams(
            needs_layout_passes=False, use_tc_tiling_on_sc=False
        ),
        scratch_types=[
            pltpu.VMEM((N_BINS,), jnp.float32),
            pltpu.VMEM((K_PER, N_BINS), jnp.float32),
            pltpu.VMEM((N_COLS, I_PER), jnp.float32),
            pltpu.VMEM((2, K_PER, I_PER), jnp.float32),
            pltpu.SemaphoreType.DMA,
            pltpu.SemaphoreType.DMA,
        ],
    )(xt, bins_pad, tabt)


def kernel(x, bins, table):
    xt = x.T
    tabt = table.T
    bins_pad = jnp.concatenate([bins, bins[-1:]])
    out_t = _embed_lookup(xt, bins_pad, tabt)
    return jnp.transpose(out_t, (2, 0, 1))


# speedup vs baseline: 129.4322x; 1.3771x over previous
import jax
import jax.numpy as jnp
from jax import lax
from jax.experimental import pallas as pl
from jax.experimental.pallas import tpu as pltpu
from jax.experimental.pallas import tpu_sc as plsc

MIN_VALUE = -4.0
MAX_VALUE = 4.0
N_BINS = 2048
EMBED_DIMS = 64
N_ROWS = 4096
N_COLS = 100
LANES = 16

K_BLOCKS = 2
K_PER = EMBED_DIMS // K_BLOCKS
I_BLOCKS = 16
I_PER = N_ROWS // I_BLOCKS
GROUPS = I_PER // LANES

INV_STEP = (N_BINS - 2) / (MAX_VALUE - MIN_VALUE)
MAX_K = float(N_BINS - 3)


def _sc_body_final(xt_hbm, bins_hbm, tabt_hbm, out_hbm, bins_v, tab_v, x_v,
                   stage_v, sem0, sem1):
    wid = lax.axis_index("s") * 2 + lax.axis_index("c")
    ib = wid // K_BLOCKS
    kb = wid % K_BLOCKS
    i0 = ib * I_PER
    k0 = kb * K_PER

    pltpu.sync_copy(bins_hbm, bins_v)
    pltpu.sync_copy(tabt_hbm.at[pl.ds(k0, K_PER), :], tab_v)
    pltpu.sync_copy(xt_hbm.at[:, pl.ds(i0, I_PER)], x_v)

    sems = (sem0, sem1)

    def column(j, p):
        def group(g, _):
            xv = x_v[j, pl.ds(g * LANES, LANES)]
            t = (xv - MIN_VALUE) * INV_STEP
            t = jnp.minimum(jnp.maximum(t, 0.0), MAX_K)
            kk = t.astype(jnp.int32)
            bk = plsc.load_gather(bins_v, [kk])
            bk1 = plsc.load_gather(bins_v, [kk + 1])
            idx = kk + (xv > bk).astype(jnp.int32) + (xv > bk1).astype(jnp.int32)
            for k in range(K_PER):
                val = plsc.load_gather(tab_v,
                                       [jnp.full((LANES,), k, jnp.int32), idx])
                stage_v[p, k, pl.ds(g * LANES, LANES)] = val
            return 0

        lax.fori_loop(0, GROUPS, group, 0)

    def write(j, p):
        return pltpu.async_copy(
            stage_v.at[p], out_hbm.at[j, pl.ds(k0, K_PER), pl.ds(i0, I_PER)],
            sems[p])

    column(0, 0)
    cp0 = write(0, 0)
    column(1, 1)
    cp1 = write(1, 1)

    def step(j2, _):
        j = j2 * 2 + 2
        cp0.wait()
        column(j, 0)
        write(j, 0)
        cp1.wait()
        column(j + 1, 1)
        write(j + 1, 1)
        return 0

    lax.fori_loop(0, (N_COLS - 2) // 2, step, 0)
    cp0.wait()
    cp1.wait()


@jax.jit
def _embed_lookup(xt, bins_pad, tabt):
    mesh = plsc.VectorSubcoreMesh(core_axis_name="c", subcore_axis_name="s")
    return pl.kernel(
        _sc_body_final,
        out_type=jax.ShapeDtypeStruct((N_COLS, EMBED_DIMS, N_ROWS), jnp.float32),
        mesh=mesh,
        compiler_params=pltpu.CompilerParams(
            needs_layout_passes=False, use_tc_tiling_on_sc=True
        ),
        scratch_types=[
            pltpu.VMEM((N_BINS,), jnp.float32),
            pltpu.VMEM((K_PER, N_BINS), jnp.float32),
            pltpu.VMEM((N_COLS, I_PER), jnp.float32),
            pltpu.VMEM((2, K_PER, I_PER), jnp.float32),
            pltpu.SemaphoreType.DMA,
            pltpu.SemaphoreType.DMA,
        ],
    )(xt, bins_pad, tabt)


def kernel(x, bins, table):
    xt = x.T
    tabt = table.T
    bins_pad = jnp.concatenate([bins, bins[-1:]])
    out_t = _embed_lookup(xt, bins_pad, tabt)
    return jnp.transpose(out_t, (2, 0, 1))


# speedup vs baseline: 331.5659x; 2.5617x over previous
import jax
import jax.numpy as jnp
from jax import lax
from jax.experimental import pallas as pl
from jax.experimental.pallas import tpu as pltpu
from jax.experimental.pallas import tpu_sc as plsc

MIN_VALUE = -4.0
MAX_VALUE = 4.0
N_BINS = 2048
EMBED_DIMS = 64
N_ROWS = 4096
N_COLS = 100
LANES = 16

K_BLOCKS = 2
K_PER = EMBED_DIMS // K_BLOCKS
I_BLOCKS = 16
I_PER = N_ROWS // I_BLOCKS
GROUPS = I_PER // LANES

INV_STEP = (N_BINS - 2) / (MAX_VALUE - MIN_VALUE)
MAX_K = float(N_BINS - 3)


def _sc_body_final(xt_hbm, bins_hbm, tabt_hbm, out_hbm, bins_v, tab_v, x_v,
                   stage_v, sem0, sem1):
    wid = lax.axis_index("s") * 2 + lax.axis_index("c")
    ib = wid // K_BLOCKS
    kb = wid % K_BLOCKS
    i0 = ib * I_PER
    k0 = kb * K_PER

    pltpu.sync_copy(bins_hbm, bins_v)
    pltpu.sync_copy(tabt_hbm.at[pl.ds(k0, K_PER), :], tab_v)
    pltpu.sync_copy(xt_hbm.at[:, pl.ds(i0, I_PER)], x_v)

    sems = (sem0, sem1)

    def column(j, p):
        @plsc.parallel_loop(0, GROUPS, unroll=2)
        def group(g):
            xv = x_v[j, pl.ds(g * LANES, LANES)]
            t = (xv - MIN_VALUE) * INV_STEP
            t = jnp.minimum(jnp.maximum(t, 0.0), MAX_K)
            kk = t.astype(jnp.int32)
            bk = plsc.load_gather(bins_v, [kk])
            bk1 = plsc.load_gather(bins_v, [kk + 1])
            idx = kk + (xv > bk).astype(jnp.int32) + (xv > bk1).astype(jnp.int32)
            for kq in range(K_PER // 8):
                vals = [
                    plsc.load_gather(
                        tab_v, [jnp.full((LANES,), kq * 8 + t_, jnp.int32), idx])
                    for t_ in range(8)
                ]
                for t_ in range(8):
                    stage_v[p, kq * 8 + t_, pl.ds(g * LANES, LANES)] = vals[t_]

    def write(j, p):
        return pltpu.async_copy(
            stage_v.at[p], out_hbm.at[j, pl.ds(k0, K_PER), pl.ds(i0, I_PER)],
            sems[p])

    column(0, 0)
    cp0 = write(0, 0)
    column(1, 1)
    cp1 = write(1, 1)

    def step(j2, _):
        j = j2 * 2 + 2
        cp0.wait()
        column(j, 0)
        write(j, 0)
        cp1.wait()
        column(j + 1, 1)
        write(j + 1, 1)
        return 0

    lax.fori_loop(0, (N_COLS - 2) // 2, step, 0)
    cp0.wait()
    cp1.wait()


@jax.jit
def _embed_lookup(xt, bins_pad, tabt):
    mesh = plsc.VectorSubcoreMesh(core_axis_name="c", subcore_axis_name="s")
    return pl.kernel(
        _sc_body_final,
        out_type=jax.ShapeDtypeStruct((N_COLS, EMBED_DIMS, N_ROWS), jnp.float32),
        mesh=mesh,
        compiler_params=pltpu.CompilerParams(
            needs_layout_passes=False, use_tc_tiling_on_sc=True
        ),
        scratch_types=[
            pltpu.VMEM((N_BINS,), jnp.float32),
            pltpu.VMEM((K_PER, N_BINS), jnp.float32),
            pltpu.VMEM((N_COLS, I_PER), jnp.float32),
            pltpu.VMEM((2, K_PER, I_PER), jnp.float32),
            pltpu.SemaphoreType.DMA,
            pltpu.SemaphoreType.DMA,
        ],
    )(xt, bins_pad, tabt)


def kernel(x, bins, table):
    xt = x.T
    tabt = table.T
    bins_pad = jnp.concatenate([bins, bins[-1:]])
    out_t = _embed_lookup(xt, bins_pad, tabt)
    return jnp.transpose(out_t, (2, 0, 1))
